# baseline (speedup 1.0000x reference)
import numpy as np

import jax
import jax.numpy as jnp
from jax.experimental import pallas as pl
from jax.experimental.pallas import tpu as pltpu

_EPS = 1e-3
_F32 = jnp.float32
_BF16 = jnp.bfloat16
_PACK = 4


def _consts(P, C, OUT):
    PC = P * C
    ip = np.arange(PC) // C
    ic = np.arange(PC) % C
    one_p = (ip[:, None] == np.arange(P)[None, :]).astype(np.float32)
    one_c = (ic[:, None] == np.arange(C)[None, :]).astype(np.float32)
    m_mean = np.concatenate([one_p / C, one_c / P], axis=1)
    s_cand = np.zeros((PC, 8 * P), np.float32)
    for t in range(8):
        for p in range(P):
            s_cand[p * C + (C - 8) + t, t * P + p] = 1.0
    b_p = one_p.T
    w_mask4 = np.kron(np.eye(_PACK, dtype=np.float32),
                      np.ones((C, OUT), np.float32))
    return (jnp.asarray(m_mean, _BF16), jnp.asarray(s_cand, _BF16),
            jnp.asarray(b_p, _BF16), jnp.asarray(w_mask4, _BF16))


def _fused_kernel(P, C, OUT):
    PC = P * C
    G = P // _PACK
    GW = _PACK * C

    def body(x_ref, mmean_ref, scand_ref, bp_ref, w4_ref,
             w1_ref, b1_ref, w2_ref, b2_ref, mm_ref, st_ref):
        tn = x_ref.shape[0]
        xb = x_ref[...]

        means = jnp.dot(xb, mmean_ref[...], preferred_element_type=_F32)

        r = xb
        for s in (C // 2, C // 4, C // 8):
            r = jnp.maximum(r, jnp.concatenate([r[:, :s], r[:, :-s]], axis=1))
        pm8 = jnp.dot(r, scand_ref[...], preferred_element_type=_F32)
        w = 4 * P
        while w >= P:
            pm8 = jnp.maximum(pm8[:, :w], pm8[:, w:2 * w])
            w //= 2
        pmax = pm8

        m = xb
        w = PC // 2
        while w >= C:
            m = jnp.maximum(m[:, :w], m[:, w:2 * w])
            w //= 2
        cmax = m.astype(_F32)

        u = jnp.concatenate(
            [jnp.concatenate([pmax, cmax], axis=1), means], axis=0)
        h = jnp.maximum(jnp.dot(u, w1_ref[...],
                                preferred_element_type=_F32) + b1_ref[...], 0.0)
        a = jnp.dot(h, w2_ref[...], preferred_element_type=_F32) + b2_ref[...]
        scales = a[:tn] + a[tn:]

        sp_b = jnp.dot(scales[:, :P].astype(_BF16), bp_ref[...],
                       preferred_element_type=_F32)
        sc_b = jnp.concatenate([scales[:, P:]] * P, axis=1)
        g = jax.nn.sigmoid(sp_b * sc_b)
        xg = xb * g.astype(_BF16)

        w4 = w4_ref[...]
        vmax = vmin = None
        ssum = qsum = None
        for j in range(G):
            yg = jnp.dot(xg[:, j * GW:(j + 1) * GW], w4,
                         preferred_element_type=_F32)
            gmax, gmin = yg, yg
            w = GW // 2
            while w >= OUT:
                gmax = jnp.maximum(gmax[:, :w], gmax[:, w:2 * w])
                gmin = jnp.minimum(gmin[:, :w], gmin[:, w:2 * w])
                w //= 2
            vmax = gmax if vmax is None else jnp.maximum(vmax, gmax)
            vmin = gmin if vmin is None else jnp.minimum(vmin, gmin)
            ssum = yg if ssum is None else ssum + yg
            qsum = yg * yg if qsum is None else qsum + yg * yg
        mm_ref[...] = jnp.concatenate([vmax, vmin], axis=1)

        sq = jnp.concatenate([ssum, qsum], axis=0)
        w = GW // 2
        while w >= OUT:
            sq = sq[:, :w] + sq[:, w:2 * w]
            w //= 2
        tsum = jnp.sum(sq[:tn], axis=0, keepdims=True)
        tsq = jnp.sum(sq[tn:], axis=0, keepdims=True)
        st_ref[...] = jnp.concatenate([tsum, tsq], axis=1)[None]

    return body


def kernel(x, w1p, b1p, w2p, b2p, w1c, b1c, w2c, b2c, w_lin, gamma, beta):
    N, P, C = x.shape
    OUT = w_lin.shape[1]
    HP, HC = w1p.shape[1], w1c.shape[1]
    PC, NU, NH = P * C, P + C, HP + HC

    tn = 256
    while N % tn:
        tn //= 2
    grid_n = N // tn

    m_mean, s_cand, b_p, w_mask4 = _consts(P, C, OUT)

    w1 = jnp.zeros((NU, NH), _F32).at[:P, :HP].set(w1p).at[P:, HP:].set(w1c)
    b1 = jnp.concatenate([b1p, b1c], axis=1)
    w2 = jnp.zeros((NH, NU), _F32).at[:HP, :P].set(w2p).at[HP:, P:].set(w2c)
    b2 = jnp.concatenate([b2p, b2c], axis=1)

    w4 = w_mask4 * jnp.tile(w_lin.astype(_BF16), (_PACK, _PACK))

    x_flat = x.reshape(N, PC).astype(_BF16)

    mm, stats = pl.pallas_call(
        _fused_kernel(P, C, OUT),
        out_shape=(
            jax.ShapeDtypeStruct((N, 2 * OUT), _F32),
            jax.ShapeDtypeStruct((grid_n, 1, 2 * OUT), _F32),
        ),
        grid=(grid_n,),
        in_specs=[
            pl.BlockSpec((tn, PC), lambda i: (i, 0)),
            pl.BlockSpec((PC, NU), lambda i: (0, 0)),
            pl.BlockSpec((PC, 8 * P), lambda i: (0, 0)),
            pl.BlockSpec((P, PC), lambda i: (0, 0)),
            pl.BlockSpec((_PACK * C, _PACK * OUT), lambda i: (0, 0)),
            pl.BlockSpec((NU, NH), lambda i: (0, 0)),
            pl.BlockSpec((1, NH), lambda i: (0, 0)),
            pl.BlockSpec((NH, NU), lambda i: (0, 0)),
            pl.BlockSpec((1, NU), lambda i: (0, 0)),
        ],
        out_specs=(
            pl.BlockSpec((tn, 2 * OUT), lambda i: (i, 0)),
            pl.BlockSpec((1, 1, 2 * OUT), lambda i: (i, 0, 0)),
        ),
        compiler_params=pltpu.CompilerParams(
            dimension_semantics=("parallel",),
            vmem_limit_bytes=64 * 1024 * 1024,
        ),
    )(x_flat, m_mean, s_cand, b_p, w4, w1, b1, w2, b2)

    npts = tn * P
    tmean = stats[:, 0, :OUT] / npts
    tsq = stats[:, 0, OUT:] / npts
    mean = jnp.mean(tmean, axis=0)
    var = jnp.mean(tsq, axis=0) - jnp.square(mean)
    scale = gamma.reshape(-1) * jax.lax.rsqrt(var + _EPS)
    shift = beta.reshape(-1) - mean * scale
    pre = jnp.where(scale >= 0.0, mm[:, :OUT], mm[:, OUT:]) * scale + shift
    return jnp.maximum(pre, 0.0).reshape(N, 1, OUT)


# speedup vs baseline: 1.0563x; 1.0563x over previous
import numpy as np

import jax
import jax.numpy as jnp
from jax.experimental import pallas as pl
from jax.experimental.pallas import tpu as pltpu

_EPS = 1e-3
_F32 = jnp.float32
_BF16 = jnp.bfloat16
_PACK = 4


def _consts(P, C, OUT):
    PC = P * C
    ip = np.arange(PC) // C
    ic = np.arange(PC) % C
    one_p = (ip[:, None] == np.arange(P)[None, :]).astype(np.float32)
    one_c = (ic[:, None] == np.arange(C)[None, :]).astype(np.float32)
    m_mean = np.concatenate([one_p / C, one_c / P], axis=1)
    s_cand = np.zeros((PC, 8 * P), np.float32)
    for t in range(8):
        for p in range(P):
            s_cand[p * C + (C - 8) + t, t * P + p] = 1.0
    b_p = one_p.T
    w_mask4 = np.kron(np.eye(_PACK, dtype=np.float32),
                      np.ones((C, OUT), np.float32))
    return (jnp.asarray(m_mean, _BF16), jnp.asarray(s_cand, _BF16),
            jnp.asarray(b_p, _BF16), jnp.asarray(w_mask4, _BF16))


def _fused_kernel(P, C, OUT):
    PC = P * C
    G = P // _PACK
    GW = _PACK * C

    def body(x_ref, mmean_ref, scand_ref, bp_ref, w4_ref,
             w1_ref, b1_ref, w2_ref, b2_ref, mm_ref, st_ref):
        tn = x_ref.shape[0]
        xb = x_ref[...].astype(_BF16)

        means = jnp.dot(xb, mmean_ref[...], preferred_element_type=_F32)

        r = xb
        for s in (C // 2, C // 4, C // 8):
            r = jnp.maximum(r, jnp.concatenate([r[:, :s], r[:, :-s]], axis=1))
        pm8 = jnp.dot(r, scand_ref[...], preferred_element_type=_F32)
        w = 4 * P
        while w >= P:
            pm8 = jnp.maximum(pm8[:, :w], pm8[:, w:2 * w])
            w //= 2
        pmax = pm8

        m = xb
        w = PC // 2
        while w >= C:
            m = jnp.maximum(m[:, :w], m[:, w:2 * w])
            w //= 2
        cmax = m.astype(_F32)

        u = jnp.concatenate(
            [jnp.concatenate([pmax, cmax], axis=1), means], axis=0)
        h = jnp.maximum(jnp.dot(u, w1_ref[...],
                                preferred_element_type=_F32) + b1_ref[...], 0.0)
        a = jnp.dot(h, w2_ref[...], preferred_element_type=_F32) + b2_ref[...]
        scales = a[:tn] + a[tn:]

        sp_b = jnp.dot(scales[:, :P].astype(_BF16), bp_ref[...],
                       preferred_element_type=_F32)
        sc_b = jnp.concatenate([scales[:, P:]] * P, axis=1)
        g = jax.nn.sigmoid(sp_b * sc_b)
        xg = xb * g.astype(_BF16)

        w4 = w4_ref[...]
        vmax = vmin = None
        ssum = qsum = None
        for j in range(G):
            yg = jnp.dot(xg[:, j * GW:(j + 1) * GW], w4,
                         preferred_element_type=_F32)
            gmax, gmin = yg, yg
            w = GW // 2
            while w >= OUT:
                gmax = jnp.maximum(gmax[:, :w], gmax[:, w:2 * w])
                gmin = jnp.minimum(gmin[:, :w], gmin[:, w:2 * w])
                w //= 2
            vmax = gmax if vmax is None else jnp.maximum(vmax, gmax)
            vmin = gmin if vmin is None else jnp.minimum(vmin, gmin)
            ssum = yg if ssum is None else ssum + yg
            qsum = yg * yg if qsum is None else qsum + yg * yg
        mm_ref[...] = jnp.concatenate([vmax, vmin], axis=1)

        sq = jnp.concatenate([ssum, qsum], axis=0)
        w = GW // 2
        while w >= OUT:
            sq = sq[:, :w] + sq[:, w:2 * w]
            w //= 2
        tsum = jnp.sum(sq[:tn], axis=0, keepdims=True)
        tsq = jnp.sum(sq[tn:], axis=0, keepdims=True)
        st_ref[...] = jnp.concatenate([tsum, tsq], axis=1)[None]

    return body


def kernel(x, w1p, b1p, w2p, b2p, w1c, b1c, w2c, b2c, w_lin, gamma, beta):
    N, P, C = x.shape
    OUT = w_lin.shape[1]
    HP, HC = w1p.shape[1], w1c.shape[1]
    PC, NU, NH = P * C, P + C, HP + HC

    tn = 256
    while N % tn:
        tn //= 2
    grid_n = N // tn

    m_mean, s_cand, b_p, w_mask4 = _consts(P, C, OUT)

    w1 = jnp.zeros((NU, NH), _F32).at[:P, :HP].set(w1p).at[P:, HP:].set(w1c)
    b1 = jnp.concatenate([b1p, b1c], axis=1)
    w2 = jnp.zeros((NH, NU), _F32).at[:HP, :P].set(w2p).at[HP:, P:].set(w2c)
    b2 = jnp.concatenate([b2p, b2c], axis=1)

    w4 = w_mask4 * jnp.tile(w_lin.astype(_BF16), (_PACK, _PACK))

    x_flat = x.reshape(N, PC)

    mm, stats = pl.pallas_call(
        _fused_kernel(P, C, OUT),
        out_shape=(
            jax.ShapeDtypeStruct((N, 2 * OUT), _F32),
            jax.ShapeDtypeStruct((grid_n, 1, 2 * OUT), _F32),
        ),
        grid=(grid_n,),
        in_specs=[
            pl.BlockSpec((tn, PC), lambda i: (i, 0)),
            pl.BlockSpec((PC, NU), lambda i: (0, 0)),
            pl.BlockSpec((PC, 8 * P), lambda i: (0, 0)),
            pl.BlockSpec((P, PC), lambda i: (0, 0)),
            pl.BlockSpec((_PACK * C, _PACK * OUT), lambda i: (0, 0)),
            pl.BlockSpec((NU, NH), lambda i: (0, 0)),
            pl.BlockSpec((1, NH), lambda i: (0, 0)),
            pl.BlockSpec((NH, NU), lambda i: (0, 0)),
            pl.BlockSpec((1, NU), lambda i: (0, 0)),
        ],
        out_specs=(
            pl.BlockSpec((tn, 2 * OUT), lambda i: (i, 0)),
            pl.BlockSpec((1, 1, 2 * OUT), lambda i: (i, 0, 0)),
        ),
        compiler_params=pltpu.CompilerParams(
            dimension_semantics=("parallel",),
            vmem_limit_bytes=64 * 1024 * 1024,
        ),
    )(x_flat, m_mean, s_cand, b_p, w4, w1, b1, w2, b2)

    npts = tn * P
    tmean = stats[:, 0, :OUT] / npts
    tsq = stats[:, 0, OUT:] / npts
    mean = jnp.mean(tmean, axis=0)
    var = jnp.mean(tsq, axis=0) - jnp.square(mean)
    scale = gamma.reshape(-1) * jax.lax.rsqrt(var + _EPS)
    shift = beta.reshape(-1) - mean * scale
    pre = jnp.where(scale >= 0.0, mm[:, :OUT], mm[:, OUT:]) * scale + shift
    return jnp.maximum(pre, 0.0).reshape(N, 1, OUT)
